# baseline (speedup 1.0000x reference)
import jax
import jax.numpy as jnp
from jax.experimental import pallas as pl
from jax.experimental.pallas import tpu as pltpu


def _elu_bias(acc, b):
    x = acc + b
    return jnp.where(x > 0, x, jnp.exp(jnp.minimum(x, 0.0)) - 1.0)


def _gcn_body(x_ref, w_ref, b_ref, adj_a_ref, adj_b_ref, o_ref, sup_ref):
    @pl.when(pl.program_id(0) == 0)
    def _build_support():
        sup_ref[...] = jnp.dot(x_ref[...], w_ref[...],
                               preferred_element_type=jnp.float32)

    bm = adj_a_ref.shape[0]
    sup = sup_ref[...]
    b = b_ref[...]
    acc_a = jnp.dot(adj_a_ref[...], sup, preferred_element_type=jnp.float32)
    o_ref[0:bm, :] = _elu_bias(acc_a, b)
    acc_b = jnp.dot(adj_b_ref[...], sup, preferred_element_type=jnp.float32)
    o_ref[bm:2 * bm, :] = _elu_bias(acc_b, b)


def kernel(inputs, adj, weight, bias):
    n, in_f = inputs.shape
    out_f = weight.shape[1]
    bm = 200
    bias2d = bias.reshape(1, out_f)
    out = pl.pallas_call(
        _gcn_body,
        grid=(n // (2 * bm),),
        in_specs=[
            pl.BlockSpec((n, in_f), lambda m: (0, 0)),
            pl.BlockSpec((in_f, out_f), lambda m: (0, 0)),
            pl.BlockSpec((1, out_f), lambda m: (0, 0)),
            pl.BlockSpec((bm, n), lambda m: (2 * m, 0)),
            pl.BlockSpec((bm, n), lambda m: (2 * m + 1, 0)),
        ],
        out_specs=pl.BlockSpec((2 * bm, out_f), lambda m: (m, 0)),
        out_shape=jax.ShapeDtypeStruct((n, out_f), jnp.float32),
        scratch_shapes=[pltpu.VMEM((n, out_f), jnp.float32)],
        compiler_params=pltpu.CompilerParams(
            dimension_semantics=("arbitrary",),
        ),
    )(inputs, weight, bias2d, adj, adj)
    return out


# speedup vs baseline: 1.0241x; 1.0241x over previous
import jax
import jax.numpy as jnp
from jax.experimental import pallas as pl
from jax.experimental.pallas import tpu as pltpu


def _gcn_body(x_ref, w_ref, b_ref, adj_ref, o_ref, sup_ref):
    @pl.when(pl.program_id(0) == 0)
    def _build_support():
        sup_ref[...] = jnp.dot(x_ref[...], w_ref[...],
                               preferred_element_type=jnp.float32)

    acc = jnp.dot(adj_ref[...], sup_ref[...],
                  preferred_element_type=jnp.float32)
    x = acc + b_ref[...]
    o_ref[...] = jnp.where(x > 0, x, jnp.exp(jnp.minimum(x, 0.0)) - 1.0)


def kernel(inputs, adj, weight, bias):
    n, in_f = inputs.shape
    out_f = weight.shape[1]
    bm = 400
    bias2d = bias.reshape(1, out_f)
    out = pl.pallas_call(
        _gcn_body,
        grid=(n // bm,),
        in_specs=[
            pl.BlockSpec((n, in_f), lambda m: (0, 0)),
            pl.BlockSpec((in_f, out_f), lambda m: (0, 0)),
            pl.BlockSpec((1, out_f), lambda m: (0, 0)),
            pl.BlockSpec((bm, n), lambda m: (m, 0)),
        ],
        out_specs=pl.BlockSpec((bm, out_f), lambda m: (m, 0)),
        out_shape=jax.ShapeDtypeStruct((n, out_f), jnp.float32),
        scratch_shapes=[pltpu.VMEM((n, out_f), jnp.float32)],
        compiler_params=pltpu.CompilerParams(
            dimension_semantics=("arbitrary",),
        ),
    )(inputs, weight, bias2d, adj)
    return out
